# Initial kernel scaffold; baseline (speedup 1.0000x reference)
import jax
import jax.numpy as jnp
from jax.experimental import pallas as pl


def kernel(op_feats, nconfig_feats, emb, pre_W1, pre_b1, pre_W2, pre_b2, gc1_W1, gc1_b1, gc1_W2, gc1_b2, gc2_W1, gc2_b1, gc2_W2, gc2_b2, post_W1, post_W2, op_ids, selected, feed_edges, sampled_feed_edges, config_dst, sampled_config_dst, graph_id_op, graph_id_config):
    raise NotImplementedError("write your pallas kernel here")


# speedup vs baseline: 87.0710x; 87.0710x over previous
import functools

import jax
import jax.numpy as jnp
from jax import lax
from jax.experimental import pallas as pl
from jax.experimental.pallas import tpu as pltpu
from jax.experimental.pallas import tpu_sc as plsc

N_OPS = 10000
NC = 1000
K = 5
G = 8
ALPHA = 0.2
NPAD = 10240
CPAD = 1024
EBLK = 128
NW = 32
ROWS_W = NPAD // NW
ROWS_SUB = NPAD // 16
CF_W = 128
ZW = 160
DW = 16


def _leaky(x):
    return jnp.where(x >= 0, x, ALPHA * x)


def _mesh():
    return plsc.VectorSubcoreMesh(core_axis_name="c", subcore_axis_name="s")


_SC_PARAMS = pltpu.CompilerParams(use_tc_tiling_on_sc=False)


def _fill2d(ref, n, m, val, dtype):
    chunks = m // 16
    v = jnp.full((16,), val, dtype)

    def body(t, carry):
        r = t // chunks
        c = (t % chunks) * 16
        ref[r, pl.ds(c, 16)] = v
        return carry

    lax.fori_loop(0, n * chunks, body, 0)


def _sc_cf_body(cdst_hbm, cfrows_hbm, cf_out, cf_acc, cidx_v, crow_v, zbuf_v):
    core = lax.axis_index("c")
    sub = lax.axis_index("s")
    wid = sub * 2 + core

    _fill2d(zbuf_v, 128, CF_W, 0.0, jnp.float32)
    r0 = sub * ROWS_SUB
    for j in range(ROWS_SUB // 128):
        pltpu.sync_copy(zbuf_v, cf_acc.at[pl.ds(r0 + j * 128, 128)])
    plsc.subcore_barrier()

    pltpu.sync_copy(cdst_hbm.at[pl.ds(wid * 32, 32)], cidx_v)
    pltpu.sync_copy(cfrows_hbm.at[pl.ds(wid * 32, 32)], crow_v)
    pltpu.sync_copy(crow_v, cf_acc.at[cidx_v], add=True)
    plsc.subcore_barrier()

    pltpu.sync_copy(cf_acc.at[pl.ds(r0, ROWS_SUB)],
                    cf_out.at[core, pl.ds(r0, ROWS_SUB)])


def _sc_cf(cdst, cfrows):
    f = pl.kernel(
        _sc_cf_body,
        out_type=jax.ShapeDtypeStruct((2, NPAD, CF_W), jnp.float32),
        mesh=_mesh(),
        compiler_params=_SC_PARAMS,
        scratch_types=[
            pltpu.VMEM_SHARED((NPAD, CF_W), jnp.float32),
            pltpu.VMEM((32,), jnp.int32),
            pltpu.VMEM((32, CF_W), jnp.float32),
            pltpu.VMEM((128, CF_W), jnp.float32),
        ],
    )
    return f(cdst, cfrows)


def _sc_deg_body(nblk, dstdir_hbm, deg_out, deg_acc, idx_v, ones_v):
    core = lax.axis_index("c")
    sub = lax.axis_index("s")
    wid = sub * 2 + core

    _fill2d(ones_v, EBLK, DW, 0.0, jnp.float32)
    r0 = sub * ROWS_SUB
    for j in range(ROWS_SUB // EBLK):
        pltpu.sync_copy(ones_v, deg_acc.at[pl.ds(r0 + j * EBLK, EBLK)])
    _fill2d(ones_v, EBLK, DW, 1.0, jnp.float32)
    plsc.subcore_barrier()

    ebase = wid * nblk * EBLK

    def body(j, carry):
        pltpu.sync_copy(dstdir_hbm.at[pl.ds(ebase + j * EBLK, EBLK)], idx_v)
        pltpu.sync_copy(ones_v, deg_acc.at[idx_v], add=True)
        return carry

    lax.fori_loop(0, nblk, body, 0)
    plsc.subcore_barrier()

    pltpu.sync_copy(deg_acc.at[pl.ds(r0, ROWS_SUB)],
                    deg_out.at[core, pl.ds(r0, ROWS_SUB)])


def _sc_deg(dstdir):
    e_pad = dstdir.shape[0]
    nblk = e_pad // (NW * EBLK)
    body = functools.partial(_sc_deg_body, nblk)
    f = pl.kernel(
        body,
        out_type=jax.ShapeDtypeStruct((2, NPAD, DW), jnp.float32),
        mesh=_mesh(),
        compiler_params=_SC_PARAMS,
        scratch_types=[
            pltpu.VMEM_SHARED((NPAD, DW), jnp.float32),
            pltpu.VMEM((EBLK,), jnp.int32),
            pltpu.VMEM((EBLK, DW), jnp.float32),
        ],
    )
    return f(dstdir)


EBLK_B = 64


def _sc_agg_body(nblk, zt_hbm, src_hbm, dst_hbm, out,
                 acc, sidx_v, didx_v, rows_v, sem):
    core = lax.axis_index("c")
    sub = lax.axis_index("s")
    wid = sub * 2 + core

    _fill2d(rows_v, EBLK_B, ZW, 0.0, jnp.float32)
    r0 = sub * ROWS_SUB
    for j in range(ROWS_SUB // EBLK_B):
        pltpu.sync_copy(rows_v, acc.at[pl.ds(r0 + j * EBLK_B, EBLK_B)])
    plsc.subcore_barrier()

    ebase = wid * nblk * EBLK_B

    def body(j, carry):
        off = ebase + j * EBLK_B
        pltpu.sync_copy(src_hbm.at[pl.ds(off, EBLK_B)], sidx_v)
        pltpu.sync_copy(dst_hbm.at[pl.ds(off, EBLK_B)], didx_v)
        pltpu.async_copy(zt_hbm.at[sidx_v], rows_v, sem).wait()
        pltpu.sync_copy(rows_v, acc.at[didx_v], add=True)
        return carry

    lax.fori_loop(0, nblk, body, 0)
    plsc.subcore_barrier()

    pltpu.sync_copy(acc.at[pl.ds(r0, ROWS_SUB)],
                    out.at[core, pl.ds(r0, ROWS_SUB)])


def _sc_agg(zt, srcdir, dstdir):
    e_pad = srcdir.shape[0]
    nblk = e_pad // (NW * EBLK_B)
    body = functools.partial(_sc_agg_body, nblk)
    f = pl.kernel(
        body,
        out_type=jax.ShapeDtypeStruct((2, NPAD, ZW), jnp.float32),
        mesh=_mesh(),
        compiler_params=_SC_PARAMS,
        scratch_types=[
            pltpu.VMEM_SHARED((NPAD, ZW), jnp.float32),
            pltpu.VMEM((EBLK_B,), jnp.int32),
            pltpu.VMEM((EBLK_B,), jnp.int32),
            pltpu.VMEM((EBLK_B, ZW), jnp.float32),
            pltpu.SemaphoreType.DMA,
        ],
    )
    return f(zt, srcdir, dstdir)


def _sc_gather_body(x_hbm, cdst_hbm, out, cidx_v, rows_v, sem):
    core = lax.axis_index("c")
    sub = lax.axis_index("s")
    wid = sub * 2 + core
    pltpu.sync_copy(cdst_hbm.at[pl.ds(wid * 32, 32)], cidx_v)
    pltpu.async_copy(x_hbm.at[cidx_v], rows_v, sem).wait()
    pltpu.sync_copy(rows_v, out.at[pl.ds(wid * 32, 32)])


def _sc_gather(x, cdst):
    f = pl.kernel(
        _sc_gather_body,
        out_type=jax.ShapeDtypeStruct((CPAD, ZW), jnp.float32),
        mesh=_mesh(),
        compiler_params=_SC_PARAMS,
        scratch_types=[
            pltpu.VMEM((32,), jnp.int32),
            pltpu.VMEM((32, ZW), jnp.float32),
            pltpu.SemaphoreType.DMA,
        ],
    )
    return f(x, cdst)


NB = 8
R = NPAD // NB


def _rows(w):
    return pl.BlockSpec((R, w), lambda i: (i, 0))


def _full(shape):
    return pl.BlockSpec(shape, lambda i: tuple(0 for _ in shape))


def _t0_body(opf, ids, cf2, dg2,
             embp, wof, wem, t5, bd_pre1a, bd_pre2, pb1, pb2_5,
             bd_g1a, bd_g1b,
             x_o, zt_o, cf_o, dis_o):
    deg = 1.0 + dg2[0][:, 0:1] + dg2[1][:, 0:1]
    dis = lax.rsqrt(deg)
    cf = 100.0 * (cf2[0] + cf2[1])
    oh = jnp.where(ids[...] == lax.broadcasted_iota(jnp.int32, (R, 128), 1),
                   1.0, 0.0).astype(jnp.float32)
    e32 = jnp.dot(oh, embp[...], preferred_element_type=jnp.float32)
    p = (jnp.dot(opf[...], wof[...], preferred_element_type=jnp.float32)
         + jnp.dot(e32, wem[...], preferred_element_type=jnp.float32)
         + pb1[...])
    h = _leaky(jnp.dot(cf, bd_pre1a[...], preferred_element_type=jnp.float32)
               + jnp.dot(p, t5[...], preferred_element_type=jnp.float32))
    x = _leaky(jnp.dot(h, bd_pre2[...], preferred_element_type=jnp.float32)
               + pb2_5[...])
    z1 = (jnp.dot(cf, bd_g1a[...], preferred_element_type=jnp.float32)
          + jnp.dot(x, bd_g1b[...], preferred_element_type=jnp.float32))
    x_o[...] = x
    zt_o[...] = dis * z1
    cf_o[...] = cf
    dis_o[...] = dis


def _t0(opf, ids, cf2, dg2, embp, wof, wem, t5, bd_pre1a, bd_pre2, pb1, pb2_5,
        bd_g1a, bd_g1b):
    return pl.pallas_call(
        _t0_body,
        grid=(NB,),
        in_specs=[
            _rows(144), _rows(1),
            pl.BlockSpec((2, R, CF_W), lambda i: (0, i, 0)),
            pl.BlockSpec((2, R, DW), lambda i: (0, i, 0)),
            _full((128, 32)), _full((144, 32)), _full((32, 32)),
            _full((32, ZW)), _full((CF_W, ZW)), _full((ZW, ZW)),
            _full((1, 32)), _full((1, ZW)),
            _full((CF_W, ZW)), _full((ZW, ZW)),
        ],
        out_specs=[_rows(ZW), _rows(ZW), _rows(CF_W), _rows(1)],
        out_shape=[
            jax.ShapeDtypeStruct((NPAD, ZW), jnp.float32),
            jax.ShapeDtypeStruct((NPAD, ZW), jnp.float32),
            jax.ShapeDtypeStruct((NPAD, CF_W), jnp.float32),
            jax.ShapeDtypeStruct((NPAD, 1), jnp.float32),
        ],
    )(opf, ids, cf2, dg2, embp, wof, wem, t5,
      bd_pre1a, bd_pre2, pb1, pb2_5, bd_g1a, bd_g1b)


def _t2_body(zt1, agg, x, cf, dis,
             b1_5, bd_w2, b2_5, bd_g2a, bd_g2b,
             x2_o, zt2_o):
    a = dis[...] * (zt1[...] + agg[...][0] + agg[...][1])
    t = (jnp.dot(_leaky(a + b1_5[...]), bd_w2[...],
                 preferred_element_type=jnp.float32) + b2_5[...])
    x2 = x[...] + _leaky(t)
    z2 = (jnp.dot(cf[...], bd_g2a[...], preferred_element_type=jnp.float32)
          + jnp.dot(x2, bd_g2b[...], preferred_element_type=jnp.float32))
    x2_o[...] = x2
    zt2_o[...] = dis[...] * z2


def _t2(zt1, agg, x, cf, dis, b1_5, bd_w2, b2_5, bd_g2a, bd_g2b):
    return pl.pallas_call(
        _t2_body,
        grid=(NB,),
        in_specs=[
            _rows(ZW),
            pl.BlockSpec((2, R, ZW), lambda i: (0, i, 0)),
            _rows(ZW), _rows(CF_W), _rows(1),
            _full((1, ZW)), _full((ZW, ZW)), _full((1, ZW)),
            _full((CF_W, ZW)), _full((ZW, ZW)),
        ],
        out_specs=[_rows(ZW), _rows(ZW)],
        out_shape=[
            jax.ShapeDtypeStruct((NPAD, ZW), jnp.float32),
            jax.ShapeDtypeStruct((NPAD, ZW), jnp.float32),
        ],
    )(zt1, agg, x, cf, dis, b1_5, bd_w2, b2_5, bd_g2a, bd_g2b)


def _t3_body(zt2, agg, x2, dis, b1_5, bd_w2, b2_5, x3_o):
    a = dis[...] * (zt2[...] + agg[...][0] + agg[...][1])
    t = (jnp.dot(_leaky(a + b1_5[...]), bd_w2[...],
                 preferred_element_type=jnp.float32) + b2_5[...])
    x3_o[...] = x2[...] + _leaky(t)


def _t3(zt2, agg, x2, dis, b1_5, bd_w2, b2_5):
    return pl.pallas_call(
        _t3_body,
        grid=(NB,),
        in_specs=[
            _rows(ZW),
            pl.BlockSpec((2, R, ZW), lambda i: (0, i, 0)),
            _rows(ZW), _rows(1),
            _full((1, ZW)), _full((ZW, ZW)), _full((1, ZW)),
        ],
        out_specs=[_rows(ZW)],
        out_shape=[jax.ShapeDtypeStruct((NPAD, ZW), jnp.float32)],
    )(zt2, agg, x2, dis, b1_5, bd_w2, b2_5)


def _t3sel_body(zt2, agg, x2, dis, b1_5, bd_w2, b2_5, xfull, msk, xsel_o):
    a = dis[...] * (zt2[...] + agg[...][0] + agg[...][1])
    t = (jnp.dot(_leaky(a + b1_5[...]), bd_w2[...],
                 preferred_element_type=jnp.float32) + b2_5[...])
    x3 = x2[...] + _leaky(t)
    m = msk[...]
    xsel_o[...] = m * x3 + (1.0 - m) * xfull[...]


def _t3sel(zt2, agg, x2, dis, b1_5, bd_w2, b2_5, xfull, msk):
    return pl.pallas_call(
        _t3sel_body,
        grid=(NB,),
        in_specs=[
            _rows(ZW),
            pl.BlockSpec((2, R, ZW), lambda i: (0, i, 0)),
            _rows(ZW), _rows(1),
            _full((1, ZW)), _full((ZW, ZW)), _full((1, ZW)),
            _rows(ZW), _rows(1),
        ],
        out_specs=[_rows(ZW)],
        out_shape=[jax.ShapeDtypeStruct((NPAD, ZW), jnp.float32)],
    )(zt2, agg, x2, dis, b1_5, bd_w2, b2_5, xfull, msk)


def _t4_body(xsel, gid, cfgx, gidc, bd_pa, bd_pb, bd_pc, bd_ones, bd_p2, out):
    ohg = jnp.where(gid[...] == lax.broadcasted_iota(jnp.int32, (NPAD, 128), 1),
                    1.0, 0.0).astype(jnp.float32)
    dn = (((0,), (0,)), ((), ()))
    pooled = lax.dot_general(ohg, xsel[...], dn,
                             preferred_element_type=jnp.float32)
    counts = lax.dot_general(ohg, jnp.ones((NPAD, 8), jnp.float32), dn,
                             preferred_element_type=jnp.float32)
    ohc = jnp.where(gidc[...] == lax.broadcasted_iota(jnp.int32, (CPAD, 128), 1),
                    1.0, 0.0).astype(jnp.float32)
    pooledc = lax.dot_general(ohc, cfgx[...], dn,
                              preferred_element_type=jnp.float32)
    ps = pooled[0:8, :]
    cnt = counts[0:8, 0:1]
    pc = pooledc[0:8, :]
    mean = ps / jnp.maximum(cnt, 1.0)
    ss = jnp.dot(ps * ps, bd_ones[...], preferred_element_type=jnp.float32)
    l2s = ps * lax.rsqrt(jnp.maximum(ss, 1e-12))
    sc = jnp.dot(pc * pc, bd_ones[...], preferred_element_type=jnp.float32)
    l2c = pc * lax.rsqrt(jnp.maximum(sc, 1e-12))
    o = (jnp.dot(mean, bd_pa[...], preferred_element_type=jnp.float32)
         + jnp.dot(l2s, bd_pb[...], preferred_element_type=jnp.float32)
         + jnp.dot(l2c, bd_pc[...], preferred_element_type=jnp.float32))
    out[...] = jnp.dot(_leaky(o), bd_p2[...],
                       preferred_element_type=jnp.float32)


def _t4(xsel, gid, cfgx, gidc, bd_pa, bd_pb, bd_pc, bd_ones, bd_p2):
    return pl.pallas_call(
        _t4_body,
        grid=(1,),
        in_specs=[
            pl.BlockSpec((NPAD, ZW), lambda i: (0, 0)),
            pl.BlockSpec((NPAD, 1), lambda i: (0, 0)),
            pl.BlockSpec((CPAD, ZW), lambda i: (0, 0)),
            pl.BlockSpec((CPAD, 1), lambda i: (0, 0)),
            _full((ZW, ZW)), _full((ZW, ZW)), _full((ZW, ZW)),
            _full((ZW, ZW)), _full((ZW, K)),
        ],
        out_specs=[pl.BlockSpec((G, K), lambda i: (0, 0))],
        out_shape=[jax.ShapeDtypeStruct((G, K), jnp.float32)],
    )(xsel, gid, cfgx, gidc, bd_pa, bd_pb, bd_pc, bd_ones, bd_p2)[0]


def _bd_place(w, rs, cs, nrows, ncols):
    m = jnp.zeros((nrows, ncols), jnp.float32)
    for c in range(K):
        m = m.at[c * rs:c * rs + w.shape[0], c * cs:c * cs + w.shape[1]].set(w)
    return m


def _pad_edges(edges):
    src = edges[0].astype(jnp.int32)
    dst = edges[1].astype(jnp.int32)
    e_dir = 2 * src.shape[0]
    e_pad = -(-e_dir // (NW * EBLK)) * (NW * EBLK)
    npad = e_pad - e_dir
    padi = (N_OPS + (jnp.arange(npad, dtype=jnp.int32) % 16))
    src_dir = jnp.concatenate([src, dst, padi])
    dst_dir = jnp.concatenate([dst, src, padi])
    return src_dir, dst_dir


def kernel(op_feats, nconfig_feats, emb, pre_W1, pre_b1, pre_W2, pre_b2,
           gc1_W1, gc1_b1, gc1_W2, gc1_b2, gc2_W1, gc2_b1, gc2_W2, gc2_b2,
           post_W1, post_W2, op_ids, selected, feed_edges, sampled_feed_edges,
           config_dst, sampled_config_dst, graph_id_op, graph_id_config):
    f32 = jnp.float32

    opf = jnp.pad(op_feats, ((0, NPAD - N_OPS), (0, 4)))
    ids = jnp.pad(op_ids.astype(jnp.int32), (0, NPAD - N_OPS))[:, None]
    msk = jnp.pad(selected.astype(f32), (0, NPAD - N_OPS))[:, None]
    gid = jnp.pad(graph_id_op.astype(jnp.int32), (0, NPAD - N_OPS),
                  constant_values=127)[:, None]
    gidc = jnp.pad(graph_id_config.astype(jnp.int32), (0, CPAD - NC),
                   constant_values=127)[:, None]
    cfrows = jnp.pad(nconfig_feats, ((0, CPAD - NC), (0, 0), (0, 6)))
    cfrows = jnp.pad(cfrows.reshape(CPAD, 120), ((0, 0), (0, 8)))
    cpadi = N_OPS + (jnp.arange(CPAD - NC, dtype=jnp.int32) % 16)
    cdst_f = jnp.concatenate([config_dst.astype(jnp.int32), cpadi])
    cdst_s = jnp.concatenate([sampled_config_dst.astype(jnp.int32), cpadi])
    src_f, dst_f = _pad_edges(feed_edges)
    src_s, dst_s = _pad_edges(sampled_feed_edges)

    embp = jnp.pad(emb, ((0, 8), (0, 0)))
    w_cf, w_opf, w_emb = pre_W1[:18], pre_W1[18:158], pre_W1[158:190]
    wof = jnp.pad(w_opf, ((0, 4), (0, 0)))
    t5 = jnp.tile(jnp.eye(32, dtype=f32), (1, K))
    bd_pre1a = _bd_place(w_cf, 24, 32, CF_W, ZW)
    bd_pre2 = _bd_place(pre_W2, 32, 32, ZW, ZW)
    pb1 = pre_b1[None, :]
    pb2_5 = jnp.tile(pre_b2, K)[None, :]
    bd_g1a = _bd_place(gc1_W1[:18], 24, 32, CF_W, ZW)
    bd_g1b = _bd_place(gc1_W1[18:50], 32, 32, ZW, ZW)
    bd_g1w2 = _bd_place(gc1_W2, 32, 32, ZW, ZW)
    g1b1_5 = jnp.tile(gc1_b1, K)[None, :]
    g1b2_5 = jnp.tile(gc1_b2, K)[None, :]
    bd_g2a = _bd_place(gc2_W1[:18], 24, 32, CF_W, ZW)
    bd_g2b = _bd_place(gc2_W1[18:50], 32, 32, ZW, ZW)
    bd_g2w2 = _bd_place(gc2_W2, 32, 32, ZW, ZW)
    g2b1_5 = jnp.tile(gc2_b1, K)[None, :]
    g2b2_5 = jnp.tile(gc2_b2, K)[None, :]
    bd_pa = _bd_place(post_W1[0:32], 32, 32, ZW, ZW)
    bd_pb = _bd_place(post_W1[32:64], 32, 32, ZW, ZW)
    bd_pc = _bd_place(post_W1[64:96], 32, 32, ZW, ZW)
    bd_ones = _bd_place(jnp.ones((32, 32), f32), 32, 32, ZW, ZW)
    bd_p2 = _bd_place(post_W2, 32, 1, ZW, K)

    def path(cdst, src_dir, dst_dir):
        cf2 = _sc_cf(cdst, cfrows)
        dg2 = _sc_deg(dst_dir)
        x, zt1, cf, dis = _t0(opf, ids, cf2, dg2, embp, wof, w_emb, t5,
                              bd_pre1a, bd_pre2, pb1, pb2_5, bd_g1a, bd_g1b)
        agg1 = _sc_agg(zt1, src_dir, dst_dir)
        x2, zt2 = _t2(zt1, agg1, x, cf, dis, g1b1_5, bd_g1w2, g1b2_5,
                      bd_g2a, bd_g2b)
        agg2 = _sc_agg(zt2, src_dir, dst_dir)
        return zt2, agg2, x2, dis

    zt2f, agg2f, x2f, disf = path(cdst_f, src_f, dst_f)
    x_full = _t3(zt2f, agg2f, x2f, disf, g2b1_5, bd_g2w2, g2b2_5)[0]
    zt2s, agg2s, x2s, diss = path(cdst_s, src_s, dst_s)
    xsel = _t3sel(zt2s, agg2s, x2s, diss, g2b1_5, bd_g2w2, g2b2_5,
                  x_full, msk)[0]
    cfgx = _sc_gather(xsel, cdst_f)
    out = _t4(xsel, gid, cfgx, gidc, bd_pa, bd_pb, bd_pc, bd_ones, bd_p2)
    return out


# speedup vs baseline: 134.8730x; 1.5490x over previous
import functools

import jax
import jax.numpy as jnp
from jax import lax
from jax.experimental import pallas as pl
from jax.experimental.pallas import tpu as pltpu
from jax.experimental.pallas import tpu_sc as plsc

N_OPS = 10000
NC = 1000
K = 5
G = 8
ALPHA = 0.2
NPAD = 10240
CPAD = 1024
EBLK = 128
NW = 32
ROWS_W = NPAD // NW
ROWS_SUB = NPAD // 16
CF_W = 128
ZW = 160
DW = 16


def _leaky(x):
    return jnp.where(x >= 0, x, ALPHA * x)


def _mesh():
    return plsc.VectorSubcoreMesh(core_axis_name="c", subcore_axis_name="s")


_SC_PARAMS = pltpu.CompilerParams(use_tc_tiling_on_sc=False)


def _fill2d(ref, n, m, val, dtype):
    chunks = m // 16
    v = jnp.full((16,), val, dtype)

    def body(t, carry):
        r = t // chunks
        c = (t % chunks) * 16
        ref[r, pl.ds(c, 16)] = v
        return carry

    lax.fori_loop(0, n * chunks, body, 0)


def _sc_cf_body(cdst_hbm, cfrows_hbm, cf_out, cf_acc, cidx_v, crow_v, zbuf_v):
    core = lax.axis_index("c")
    sub = lax.axis_index("s")
    wid = sub * 2 + core

    _fill2d(zbuf_v, 128, CF_W, 0.0, jnp.float32)
    r0 = sub * ROWS_SUB
    for j in range(ROWS_SUB // 128):
        pltpu.sync_copy(zbuf_v, cf_acc.at[pl.ds(r0 + j * 128, 128)])
    plsc.subcore_barrier()

    pltpu.sync_copy(cdst_hbm.at[pl.ds(wid * 32, 32)], cidx_v)
    pltpu.sync_copy(cfrows_hbm.at[pl.ds(wid * 32, 32)], crow_v)
    pltpu.sync_copy(crow_v, cf_acc.at[cidx_v], add=True)
    plsc.subcore_barrier()

    pltpu.sync_copy(cf_acc.at[pl.ds(r0, ROWS_SUB)],
                    cf_out.at[core, pl.ds(r0, ROWS_SUB)])


def _sc_cf(cdst, cfrows):
    f = pl.kernel(
        _sc_cf_body,
        out_type=jax.ShapeDtypeStruct((2, NPAD, CF_W), jnp.float32),
        mesh=_mesh(),
        compiler_params=_SC_PARAMS,
        scratch_types=[
            pltpu.VMEM_SHARED((NPAD, CF_W), jnp.float32),
            pltpu.VMEM((32,), jnp.int32),
            pltpu.VMEM((32, CF_W), jnp.float32),
            pltpu.VMEM((128, CF_W), jnp.float32),
        ],
    )
    return f(cdst, cfrows)


def _sc_deg_body(nblk, dstdir_hbm, deg_out, deg_acc, idx_v, ones_v):
    core = lax.axis_index("c")
    sub = lax.axis_index("s")
    wid = sub * 2 + core

    _fill2d(ones_v, EBLK, DW, 0.0, jnp.float32)
    r0 = sub * ROWS_SUB
    for j in range(ROWS_SUB // EBLK):
        pltpu.sync_copy(ones_v, deg_acc.at[pl.ds(r0 + j * EBLK, EBLK)])
    _fill2d(ones_v, EBLK, DW, 1.0, jnp.float32)
    plsc.subcore_barrier()

    ebase = wid * nblk * EBLK

    def body(j, carry):
        pltpu.sync_copy(dstdir_hbm.at[pl.ds(ebase + j * EBLK, EBLK)], idx_v)
        pltpu.sync_copy(ones_v, deg_acc.at[idx_v], add=True)
        return carry

    lax.fori_loop(0, nblk, body, 0)
    plsc.subcore_barrier()

    pltpu.sync_copy(deg_acc.at[pl.ds(r0, ROWS_SUB)],
                    deg_out.at[core, pl.ds(r0, ROWS_SUB)])


def _sc_deg(dstdir):
    e_pad = dstdir.shape[0]
    nblk = e_pad // (NW * EBLK)
    body = functools.partial(_sc_deg_body, nblk)
    f = pl.kernel(
        body,
        out_type=jax.ShapeDtypeStruct((2, NPAD, DW), jnp.float32),
        mesh=_mesh(),
        compiler_params=_SC_PARAMS,
        scratch_types=[
            pltpu.VMEM_SHARED((NPAD, DW), jnp.float32),
            pltpu.VMEM((EBLK,), jnp.int32),
            pltpu.VMEM((EBLK, DW), jnp.float32),
        ],
    )
    return f(dstdir)


EBLK_B = 64


def _sc_agg_body(nblk, zt_hbm, src_hbm, dst_hbm, out,
                 acc, sidx0, didx0, rows0, sidx1, didx1, rows1, sem0, sem1):
    core = lax.axis_index("c")
    sub = lax.axis_index("s")
    wid = sub * 2 + core

    _fill2d(rows0, EBLK_B, ZW, 0.0, jnp.float32)
    r0 = sub * ROWS_SUB
    for j in range(ROWS_SUB // EBLK_B):
        pltpu.sync_copy(rows0, acc.at[pl.ds(r0 + j * EBLK_B, EBLK_B)])
    plsc.subcore_barrier()

    ebase = wid * nblk * EBLK_B

    pltpu.sync_copy(src_hbm.at[pl.ds(ebase, EBLK_B)], sidx0)
    pltpu.sync_copy(dst_hbm.at[pl.ds(ebase, EBLK_B)], didx0)
    pltpu.async_copy(zt_hbm.at[sidx0], rows0, sem0)

    def body(j2, carry):
        o1 = ebase + (2 * j2 + 1) * EBLK_B
        pltpu.sync_copy(src_hbm.at[pl.ds(o1, EBLK_B)], sidx1)
        pltpu.sync_copy(dst_hbm.at[pl.ds(o1, EBLK_B)], didx1)
        pltpu.async_copy(zt_hbm.at[sidx1], rows1, sem1)
        pltpu.make_async_copy(zt_hbm.at[sidx0], rows0, sem0).wait()
        pltpu.sync_copy(rows0, acc.at[didx0], add=True)

        @pl.when(2 * j2 + 2 < nblk)
        def _():
            o2 = ebase + (2 * j2 + 2) * EBLK_B
            pltpu.sync_copy(src_hbm.at[pl.ds(o2, EBLK_B)], sidx0)
            pltpu.sync_copy(dst_hbm.at[pl.ds(o2, EBLK_B)], didx0)
            pltpu.async_copy(zt_hbm.at[sidx0], rows0, sem0)

        pltpu.make_async_copy(zt_hbm.at[sidx1], rows1, sem1).wait()
        pltpu.sync_copy(rows1, acc.at[didx1], add=True)
        return carry

    lax.fori_loop(0, nblk // 2, body, 0)
    plsc.subcore_barrier()

    pltpu.sync_copy(acc.at[pl.ds(r0, ROWS_SUB)],
                    out.at[core, pl.ds(r0, ROWS_SUB)])


def _sc_agg(zt, srcdir, dstdir):
    e_pad = srcdir.shape[0]
    nblk = e_pad // (NW * EBLK_B)
    body = functools.partial(_sc_agg_body, nblk)
    f = pl.kernel(
        body,
        out_type=jax.ShapeDtypeStruct((2, NPAD, ZW), jnp.float32),
        mesh=_mesh(),
        compiler_params=_SC_PARAMS,
        scratch_types=[
            pltpu.VMEM_SHARED((NPAD, ZW), jnp.float32),
            pltpu.VMEM((EBLK_B,), jnp.int32),
            pltpu.VMEM((EBLK_B,), jnp.int32),
            pltpu.VMEM((EBLK_B, ZW), jnp.float32),
            pltpu.VMEM((EBLK_B,), jnp.int32),
            pltpu.VMEM((EBLK_B,), jnp.int32),
            pltpu.VMEM((EBLK_B, ZW), jnp.float32),
            pltpu.SemaphoreType.DMA,
            pltpu.SemaphoreType.DMA,
        ],
    )
    return f(zt, srcdir, dstdir)


def _sc_gather_body(x_hbm, cdst_hbm, out, cidx_v, rows_v, sem):
    core = lax.axis_index("c")
    sub = lax.axis_index("s")
    wid = sub * 2 + core
    pltpu.sync_copy(cdst_hbm.at[pl.ds(wid * 32, 32)], cidx_v)
    pltpu.async_copy(x_hbm.at[cidx_v], rows_v, sem).wait()
    pltpu.sync_copy(rows_v, out.at[pl.ds(wid * 32, 32)])


def _sc_gather(x, cdst):
    f = pl.kernel(
        _sc_gather_body,
        out_type=jax.ShapeDtypeStruct((CPAD, ZW), jnp.float32),
        mesh=_mesh(),
        compiler_params=_SC_PARAMS,
        scratch_types=[
            pltpu.VMEM((32,), jnp.int32),
            pltpu.VMEM((32, ZW), jnp.float32),
            pltpu.SemaphoreType.DMA,
        ],
    )
    return f(x, cdst)


NB = 8
R = NPAD // NB


def _rows(w):
    return pl.BlockSpec((R, w), lambda i: (i, 0))


def _full(shape):
    return pl.BlockSpec(shape, lambda i: tuple(0 for _ in shape))


def _t0_body(opf, ids, cf2, dg2,
             embp, wof, wem, t5, bd_pre1a, bd_pre2, pb1, pb2_5,
             bd_g1a, bd_g1b,
             x_o, zt_o, cf_o, dis_o):
    deg = 1.0 + dg2[0][:, 0:1] + dg2[1][:, 0:1]
    dis = lax.rsqrt(deg)
    cf = 100.0 * (cf2[0] + cf2[1])
    oh = jnp.where(ids[...] == lax.broadcasted_iota(jnp.int32, (R, 128), 1),
                   1.0, 0.0).astype(jnp.float32)
    e32 = jnp.dot(oh, embp[...], preferred_element_type=jnp.float32)
    p = (jnp.dot(opf[...], wof[...], preferred_element_type=jnp.float32)
         + jnp.dot(e32, wem[...], preferred_element_type=jnp.float32)
         + pb1[...])
    h = _leaky(jnp.dot(cf, bd_pre1a[...], preferred_element_type=jnp.float32)
               + jnp.dot(p, t5[...], preferred_element_type=jnp.float32))
    x = _leaky(jnp.dot(h, bd_pre2[...], preferred_element_type=jnp.float32)
               + pb2_5[...])
    z1 = (jnp.dot(cf, bd_g1a[...], preferred_element_type=jnp.float32)
          + jnp.dot(x, bd_g1b[...], preferred_element_type=jnp.float32))
    x_o[...] = x
    zt_o[...] = dis * z1
    cf_o[...] = cf
    dis_o[...] = dis


def _t0(opf, ids, cf2, dg2, embp, wof, wem, t5, bd_pre1a, bd_pre2, pb1, pb2_5,
        bd_g1a, bd_g1b):
    return pl.pallas_call(
        _t0_body,
        grid=(NB,),
        in_specs=[
            _rows(144), _rows(1),
            pl.BlockSpec((2, R, CF_W), lambda i: (0, i, 0)),
            pl.BlockSpec((2, R, DW), lambda i: (0, i, 0)),
            _full((128, 32)), _full((144, 32)), _full((32, 32)),
            _full((32, ZW)), _full((CF_W, ZW)), _full((ZW, ZW)),
            _full((1, 32)), _full((1, ZW)),
            _full((CF_W, ZW)), _full((ZW, ZW)),
        ],
        out_specs=[_rows(ZW), _rows(ZW), _rows(CF_W), _rows(1)],
        out_shape=[
            jax.ShapeDtypeStruct((NPAD, ZW), jnp.float32),
            jax.ShapeDtypeStruct((NPAD, ZW), jnp.float32),
            jax.ShapeDtypeStruct((NPAD, CF_W), jnp.float32),
            jax.ShapeDtypeStruct((NPAD, 1), jnp.float32),
        ],
    )(opf, ids, cf2, dg2, embp, wof, wem, t5,
      bd_pre1a, bd_pre2, pb1, pb2_5, bd_g1a, bd_g1b)


def _t2_body(zt1, agg, x, cf, dis,
             b1_5, bd_w2, b2_5, bd_g2a, bd_g2b,
             x2_o, zt2_o):
    a = dis[...] * (zt1[...] + agg[...][0] + agg[...][1])
    t = (jnp.dot(_leaky(a + b1_5[...]), bd_w2[...],
                 preferred_element_type=jnp.float32) + b2_5[...])
    x2 = x[...] + _leaky(t)
    z2 = (jnp.dot(cf[...], bd_g2a[...], preferred_element_type=jnp.float32)
          + jnp.dot(x2, bd_g2b[...], preferred_element_type=jnp.float32))
    x2_o[...] = x2
    zt2_o[...] = dis[...] * z2


def _t2(zt1, agg, x, cf, dis, b1_5, bd_w2, b2_5, bd_g2a, bd_g2b):
    return pl.pallas_call(
        _t2_body,
        grid=(NB,),
        in_specs=[
            _rows(ZW),
            pl.BlockSpec((2, R, ZW), lambda i: (0, i, 0)),
            _rows(ZW), _rows(CF_W), _rows(1),
            _full((1, ZW)), _full((ZW, ZW)), _full((1, ZW)),
            _full((CF_W, ZW)), _full((ZW, ZW)),
        ],
        out_specs=[_rows(ZW), _rows(ZW)],
        out_shape=[
            jax.ShapeDtypeStruct((NPAD, ZW), jnp.float32),
            jax.ShapeDtypeStruct((NPAD, ZW), jnp.float32),
        ],
    )(zt1, agg, x, cf, dis, b1_5, bd_w2, b2_5, bd_g2a, bd_g2b)


def _t3_body(zt2, agg, x2, dis, b1_5, bd_w2, b2_5, x3_o):
    a = dis[...] * (zt2[...] + agg[...][0] + agg[...][1])
    t = (jnp.dot(_leaky(a + b1_5[...]), bd_w2[...],
                 preferred_element_type=jnp.float32) + b2_5[...])
    x3_o[...] = x2[...] + _leaky(t)


def _t3(zt2, agg, x2, dis, b1_5, bd_w2, b2_5):
    return pl.pallas_call(
        _t3_body,
        grid=(NB,),
        in_specs=[
            _rows(ZW),
            pl.BlockSpec((2, R, ZW), lambda i: (0, i, 0)),
            _rows(ZW), _rows(1),
            _full((1, ZW)), _full((ZW, ZW)), _full((1, ZW)),
        ],
        out_specs=[_rows(ZW)],
        out_shape=[jax.ShapeDtypeStruct((NPAD, ZW), jnp.float32)],
    )(zt2, agg, x2, dis, b1_5, bd_w2, b2_5)


def _t3sel_body(zt2, agg, x2, dis, b1_5, bd_w2, b2_5, xfull, msk, xsel_o):
    a = dis[...] * (zt2[...] + agg[...][0] + agg[...][1])
    t = (jnp.dot(_leaky(a + b1_5[...]), bd_w2[...],
                 preferred_element_type=jnp.float32) + b2_5[...])
    x3 = x2[...] + _leaky(t)
    m = msk[...]
    xsel_o[...] = m * x3 + (1.0 - m) * xfull[...]


def _t3sel(zt2, agg, x2, dis, b1_5, bd_w2, b2_5, xfull, msk):
    return pl.pallas_call(
        _t3sel_body,
        grid=(NB,),
        in_specs=[
            _rows(ZW),
            pl.BlockSpec((2, R, ZW), lambda i: (0, i, 0)),
            _rows(ZW), _rows(1),
            _full((1, ZW)), _full((ZW, ZW)), _full((1, ZW)),
            _rows(ZW), _rows(1),
        ],
        out_specs=[_rows(ZW)],
        out_shape=[jax.ShapeDtypeStruct((NPAD, ZW), jnp.float32)],
    )(zt2, agg, x2, dis, b1_5, bd_w2, b2_5, xfull, msk)


def _t4_body(xsel, gid, cfgx, gidc, bd_pa, bd_pb, bd_pc, bd_ones, bd_p2, out):
    ohg = jnp.where(gid[...] == lax.broadcasted_iota(jnp.int32, (NPAD, 128), 1),
                    1.0, 0.0).astype(jnp.float32)
    dn = (((0,), (0,)), ((), ()))
    pooled = lax.dot_general(ohg, xsel[...], dn,
                             preferred_element_type=jnp.float32)
    counts = lax.dot_general(ohg, jnp.ones((NPAD, 8), jnp.float32), dn,
                             preferred_element_type=jnp.float32)
    ohc = jnp.where(gidc[...] == lax.broadcasted_iota(jnp.int32, (CPAD, 128), 1),
                    1.0, 0.0).astype(jnp.float32)
    pooledc = lax.dot_general(ohc, cfgx[...], dn,
                              preferred_element_type=jnp.float32)
    ps = pooled[0:8, :]
    cnt = counts[0:8, 0:1]
    pc = pooledc[0:8, :]
    mean = ps / jnp.maximum(cnt, 1.0)
    ss = jnp.dot(ps * ps, bd_ones[...], preferred_element_type=jnp.float32)
    l2s = ps * lax.rsqrt(jnp.maximum(ss, 1e-12))
    sc = jnp.dot(pc * pc, bd_ones[...], preferred_element_type=jnp.float32)
    l2c = pc * lax.rsqrt(jnp.maximum(sc, 1e-12))
    o = (jnp.dot(mean, bd_pa[...], preferred_element_type=jnp.float32)
         + jnp.dot(l2s, bd_pb[...], preferred_element_type=jnp.float32)
         + jnp.dot(l2c, bd_pc[...], preferred_element_type=jnp.float32))
    out[...] = jnp.dot(_leaky(o), bd_p2[...],
                       preferred_element_type=jnp.float32)


def _t4(xsel, gid, cfgx, gidc, bd_pa, bd_pb, bd_pc, bd_ones, bd_p2):
    return pl.pallas_call(
        _t4_body,
        grid=(1,),
        in_specs=[
            pl.BlockSpec((NPAD, ZW), lambda i: (0, 0)),
            pl.BlockSpec((NPAD, 1), lambda i: (0, 0)),
            pl.BlockSpec((CPAD, ZW), lambda i: (0, 0)),
            pl.BlockSpec((CPAD, 1), lambda i: (0, 0)),
            _full((ZW, ZW)), _full((ZW, ZW)), _full((ZW, ZW)),
            _full((ZW, ZW)), _full((ZW, K)),
        ],
        out_specs=[pl.BlockSpec((G, K), lambda i: (0, 0))],
        out_shape=[jax.ShapeDtypeStruct((G, K), jnp.float32)],
    )(xsel, gid, cfgx, gidc, bd_pa, bd_pb, bd_pc, bd_ones, bd_p2)[0]


def _bd_place(w, rs, cs, nrows, ncols):
    m = jnp.zeros((nrows, ncols), jnp.float32)
    for c in range(K):
        m = m.at[c * rs:c * rs + w.shape[0], c * cs:c * cs + w.shape[1]].set(w)
    return m


def _pad_edges(edges):
    src = edges[0].astype(jnp.int32)
    dst = edges[1].astype(jnp.int32)
    e_dir = 2 * src.shape[0]
    e_pad = -(-e_dir // (NW * EBLK)) * (NW * EBLK)
    npad = e_pad - e_dir
    padi = (N_OPS + (jnp.arange(npad, dtype=jnp.int32) % 16))
    src_dir = jnp.concatenate([src, dst, padi])
    dst_dir = jnp.concatenate([dst, src, padi])
    return src_dir, dst_dir


def kernel(op_feats, nconfig_feats, emb, pre_W1, pre_b1, pre_W2, pre_b2,
           gc1_W1, gc1_b1, gc1_W2, gc1_b2, gc2_W1, gc2_b1, gc2_W2, gc2_b2,
           post_W1, post_W2, op_ids, selected, feed_edges, sampled_feed_edges,
           config_dst, sampled_config_dst, graph_id_op, graph_id_config):
    f32 = jnp.float32

    opf = jnp.pad(op_feats, ((0, NPAD - N_OPS), (0, 4)))
    ids = jnp.pad(op_ids.astype(jnp.int32), (0, NPAD - N_OPS))[:, None]
    msk = jnp.pad(selected.astype(f32), (0, NPAD - N_OPS))[:, None]
    gid = jnp.pad(graph_id_op.astype(jnp.int32), (0, NPAD - N_OPS),
                  constant_values=127)[:, None]
    gidc = jnp.pad(graph_id_config.astype(jnp.int32), (0, CPAD - NC),
                   constant_values=127)[:, None]
    cfrows = jnp.pad(nconfig_feats, ((0, CPAD - NC), (0, 0), (0, 6)))
    cfrows = jnp.pad(cfrows.reshape(CPAD, 120), ((0, 0), (0, 8)))
    cpadi = N_OPS + (jnp.arange(CPAD - NC, dtype=jnp.int32) % 16)
    cdst_f = jnp.concatenate([config_dst.astype(jnp.int32), cpadi])
    cdst_s = jnp.concatenate([sampled_config_dst.astype(jnp.int32), cpadi])
    src_f, dst_f = _pad_edges(feed_edges)
    src_s, dst_s = _pad_edges(sampled_feed_edges)

    embp = jnp.pad(emb, ((0, 8), (0, 0)))
    w_cf, w_opf, w_emb = pre_W1[:18], pre_W1[18:158], pre_W1[158:190]
    wof = jnp.pad(w_opf, ((0, 4), (0, 0)))
    t5 = jnp.tile(jnp.eye(32, dtype=f32), (1, K))
    bd_pre1a = _bd_place(w_cf, 24, 32, CF_W, ZW)
    bd_pre2 = _bd_place(pre_W2, 32, 32, ZW, ZW)
    pb1 = pre_b1[None, :]
    pb2_5 = jnp.tile(pre_b2, K)[None, :]
    bd_g1a = _bd_place(gc1_W1[:18], 24, 32, CF_W, ZW)
    bd_g1b = _bd_place(gc1_W1[18:50], 32, 32, ZW, ZW)
    bd_g1w2 = _bd_place(gc1_W2, 32, 32, ZW, ZW)
    g1b1_5 = jnp.tile(gc1_b1, K)[None, :]
    g1b2_5 = jnp.tile(gc1_b2, K)[None, :]
    bd_g2a = _bd_place(gc2_W1[:18], 24, 32, CF_W, ZW)
    bd_g2b = _bd_place(gc2_W1[18:50], 32, 32, ZW, ZW)
    bd_g2w2 = _bd_place(gc2_W2, 32, 32, ZW, ZW)
    g2b1_5 = jnp.tile(gc2_b1, K)[None, :]
    g2b2_5 = jnp.tile(gc2_b2, K)[None, :]
    bd_pa = _bd_place(post_W1[0:32], 32, 32, ZW, ZW)
    bd_pb = _bd_place(post_W1[32:64], 32, 32, ZW, ZW)
    bd_pc = _bd_place(post_W1[64:96], 32, 32, ZW, ZW)
    bd_ones = _bd_place(jnp.ones((32, 32), f32), 32, 32, ZW, ZW)
    bd_p2 = _bd_place(post_W2, 32, 1, ZW, K)

    def path(cdst, src_dir, dst_dir):
        cf2 = _sc_cf(cdst, cfrows)
        dg2 = _sc_deg(dst_dir)
        x, zt1, cf, dis = _t0(opf, ids, cf2, dg2, embp, wof, w_emb, t5,
                              bd_pre1a, bd_pre2, pb1, pb2_5, bd_g1a, bd_g1b)
        agg1 = _sc_agg(zt1, src_dir, dst_dir)
        x2, zt2 = _t2(zt1, agg1, x, cf, dis, g1b1_5, bd_g1w2, g1b2_5,
                      bd_g2a, bd_g2b)
        agg2 = _sc_agg(zt2, src_dir, dst_dir)
        return zt2, agg2, x2, dis

    zt2f, agg2f, x2f, disf = path(cdst_f, src_f, dst_f)
    x_full = _t3(zt2f, agg2f, x2f, disf, g2b1_5, bd_g2w2, g2b2_5)[0]
    zt2s, agg2s, x2s, diss = path(cdst_s, src_s, dst_s)
    xsel = _t3sel(zt2s, agg2s, x2s, diss, g2b1_5, bd_g2w2, g2b2_5,
                  x_full, msk)[0]
    cfgx = _sc_gather(xsel, cdst_f)
    out = _t4(xsel, gid, cfgx, gidc, bd_pa, bd_pb, bd_pc, bd_ones, bd_p2)
    return out


# speedup vs baseline: 153.9091x; 1.1411x over previous
import functools

import jax
import jax.numpy as jnp
from jax import lax
from jax.experimental import pallas as pl
from jax.experimental.pallas import tpu as pltpu
from jax.experimental.pallas import tpu_sc as plsc

N_OPS = 10000
NC = 1000
K = 5
G = 8
ALPHA = 0.2
NPAD = 10240
CPAD = 1024
EBLK = 128
NW = 32
ROWS_W = NPAD // NW
ROWS_SUB = NPAD // 16
CF_W = 128
ZW = 160
DW = 16


def _leaky(x):
    return jnp.where(x >= 0, x, ALPHA * x)


def _mesh():
    return plsc.VectorSubcoreMesh(core_axis_name="c", subcore_axis_name="s")


_SC_PARAMS = pltpu.CompilerParams(use_tc_tiling_on_sc=False)


def _fill2d(ref, n, m, val, dtype):
    chunks = m // 16
    v = jnp.full((16,), val, dtype)

    def body(t, carry):
        r = t // chunks
        c = (t % chunks) * 16
        ref[r, pl.ds(c, 16)] = v
        return carry

    lax.fori_loop(0, n * chunks, body, 0)


def _sc_cf_body(cdst_hbm, cfrows_hbm, cf_out, cf_acc, cidx_v, crow_v, zbuf_v):
    core = lax.axis_index("c")
    sub = lax.axis_index("s")
    wid = sub * 2 + core

    _fill2d(zbuf_v, 128, CF_W, 0.0, jnp.float32)
    r0 = sub * ROWS_SUB
    for j in range(ROWS_SUB // 128):
        pltpu.sync_copy(zbuf_v, cf_acc.at[pl.ds(r0 + j * 128, 128)])
    plsc.subcore_barrier()

    pltpu.sync_copy(cdst_hbm.at[pl.ds(wid * 32, 32)], cidx_v)
    pltpu.sync_copy(cfrows_hbm.at[pl.ds(wid * 32, 32)], crow_v)
    pltpu.sync_copy(crow_v, cf_acc.at[cidx_v], add=True)
    plsc.subcore_barrier()

    pltpu.sync_copy(cf_acc.at[pl.ds(r0, ROWS_SUB)],
                    cf_out.at[core, pl.ds(r0, ROWS_SUB)])


def _sc_cf(cdst, cfrows):
    f = pl.kernel(
        _sc_cf_body,
        out_type=jax.ShapeDtypeStruct((2, NPAD, CF_W), jnp.float32),
        mesh=_mesh(),
        compiler_params=_SC_PARAMS,
        scratch_types=[
            pltpu.VMEM_SHARED((NPAD, CF_W), jnp.float32),
            pltpu.VMEM((32,), jnp.int32),
            pltpu.VMEM((32, CF_W), jnp.float32),
            pltpu.VMEM((128, CF_W), jnp.float32),
        ],
    )
    return f(cdst, cfrows)


EBLK_D = 512


def _sc_deg_body(nblk_f, nblk_s, dstf_hbm, dsts_hbm, deg_out,
                 acc_f, acc_s, idx0, idx1, ones_v, sem0, sem1):
    core = lax.axis_index("c")
    sub = lax.axis_index("s")
    wid = sub * 2 + core

    _fill2d(ones_v, EBLK_D // 4, DW, 0.0, jnp.float32)
    r0 = sub * ROWS_SUB
    for j in range(ROWS_SUB // (EBLK_D // 4)):
        pltpu.sync_copy(ones_v.at[pl.ds(0, EBLK_D // 4)],
                        acc_f.at[pl.ds(r0 + j * (EBLK_D // 4), EBLK_D // 4)])
        pltpu.sync_copy(ones_v.at[pl.ds(0, EBLK_D // 4)],
                        acc_s.at[pl.ds(r0 + j * (EBLK_D // 4), EBLK_D // 4)])
    _fill2d(ones_v, EBLK_D, DW, 1.0, jnp.float32)
    plsc.subcore_barrier()

    def run(dst_hbm, acc, nblk):
        ebase = wid * nblk * EBLK_D
        pltpu.sync_copy(dst_hbm.at[pl.ds(ebase, EBLK_D)], idx0)

        def body(j2, carry):
            pltpu.async_copy(ones_v, acc.at[idx0], sem0, add=True)
            o1 = ebase + (2 * j2 + 1) * EBLK_D
            pltpu.sync_copy(dst_hbm.at[pl.ds(o1, EBLK_D)], idx1)
            pltpu.make_async_copy(ones_v, acc.at[idx0], sem0).wait()
            pltpu.async_copy(ones_v, acc.at[idx1], sem1, add=True)

            @pl.when(2 * j2 + 2 < nblk)
            def _():
                o2 = ebase + (2 * j2 + 2) * EBLK_D
                pltpu.sync_copy(dst_hbm.at[pl.ds(o2, EBLK_D)], idx0)

            pltpu.make_async_copy(ones_v, acc.at[idx1], sem1).wait()
            return carry

        lax.fori_loop(0, nblk // 2, body, 0)

    run(dstf_hbm, acc_f, nblk_f)
    run(dsts_hbm, acc_s, nblk_s)
    plsc.subcore_barrier()

    pltpu.sync_copy(acc_f.at[pl.ds(r0, ROWS_SUB)],
                    deg_out.at[0, core, pl.ds(r0, ROWS_SUB)])
    pltpu.sync_copy(acc_s.at[pl.ds(r0, ROWS_SUB)],
                    deg_out.at[1, core, pl.ds(r0, ROWS_SUB)])


def _sc_deg2(dstdir_f, dstdir_s):
    nblk_f = dstdir_f.shape[0] // (NW * EBLK_D)
    nblk_s = dstdir_s.shape[0] // (NW * EBLK_D)
    body = functools.partial(_sc_deg_body, nblk_f, nblk_s)
    f = pl.kernel(
        body,
        out_type=jax.ShapeDtypeStruct((2, 2, NPAD, DW), jnp.float32),
        mesh=_mesh(),
        compiler_params=_SC_PARAMS,
        scratch_types=[
            pltpu.VMEM_SHARED((NPAD, DW), jnp.float32),
            pltpu.VMEM_SHARED((NPAD, DW), jnp.float32),
            pltpu.VMEM((EBLK_D,), jnp.int32),
            pltpu.VMEM((EBLK_D,), jnp.int32),
            pltpu.VMEM((EBLK_D, DW), jnp.float32),
            pltpu.SemaphoreType.DMA,
            pltpu.SemaphoreType.DMA,
        ],
    )
    return f(dstdir_f, dstdir_s)


EBLK_B = 80


def _sc_agg_body(nblk, zt_hbm, src_hbm, dst_hbm, out,
                 acc, sidx0, didx0, rows0, sidx1, didx1, rows1, sem0, sem1):
    core = lax.axis_index("c")
    sub = lax.axis_index("s")
    wid = sub * 2 + core

    _fill2d(rows0, EBLK_B, ZW, 0.0, jnp.float32)
    r0 = sub * ROWS_SUB
    for j in range(ROWS_SUB // EBLK_B):
        pltpu.sync_copy(rows0, acc.at[pl.ds(r0 + j * EBLK_B, EBLK_B)])
    plsc.subcore_barrier()

    ebase = wid * nblk * EBLK_B

    pltpu.sync_copy(src_hbm.at[pl.ds(ebase, EBLK_B)], sidx0)
    pltpu.sync_copy(dst_hbm.at[pl.ds(ebase, EBLK_B)], didx0)
    pltpu.async_copy(zt_hbm.at[sidx0], rows0, sem0)

    def body(j2, carry):
        o1 = ebase + (2 * j2 + 1) * EBLK_B
        pltpu.sync_copy(src_hbm.at[pl.ds(o1, EBLK_B)], sidx1)
        pltpu.sync_copy(dst_hbm.at[pl.ds(o1, EBLK_B)], didx1)
        pltpu.async_copy(zt_hbm.at[sidx1], rows1, sem1)
        pltpu.make_async_copy(zt_hbm.at[sidx0], rows0, sem0).wait()
        pltpu.sync_copy(rows0, acc.at[didx0], add=True)

        @pl.when(2 * j2 + 2 < nblk)
        def _():
            o2 = ebase + (2 * j2 + 2) * EBLK_B
            pltpu.sync_copy(src_hbm.at[pl.ds(o2, EBLK_B)], sidx0)
            pltpu.sync_copy(dst_hbm.at[pl.ds(o2, EBLK_B)], didx0)
            pltpu.async_copy(zt_hbm.at[sidx0], rows0, sem0)

        pltpu.make_async_copy(zt_hbm.at[sidx1], rows1, sem1).wait()
        pltpu.sync_copy(rows1, acc.at[didx1], add=True)
        return carry

    lax.fori_loop(0, nblk // 2, body, 0)
    plsc.subcore_barrier()

    pltpu.sync_copy(acc.at[pl.ds(r0, ROWS_SUB)],
                    out.at[core, pl.ds(r0, ROWS_SUB)])


def _sc_agg(zt, srcdir, dstdir):
    e_pad = srcdir.shape[0]
    nblk = e_pad // (NW * EBLK_B)
    body = functools.partial(_sc_agg_body, nblk)
    f = pl.kernel(
        body,
        out_type=jax.ShapeDtypeStruct((2, NPAD, ZW), jnp.float32),
        mesh=_mesh(),
        compiler_params=_SC_PARAMS,
        scratch_types=[
            pltpu.VMEM_SHARED((NPAD, ZW), jnp.float32),
            pltpu.VMEM((EBLK_B,), jnp.int32),
            pltpu.VMEM((EBLK_B,), jnp.int32),
            pltpu.VMEM((EBLK_B, ZW), jnp.float32),
            pltpu.VMEM((EBLK_B,), jnp.int32),
            pltpu.VMEM((EBLK_B,), jnp.int32),
            pltpu.VMEM((EBLK_B, ZW), jnp.float32),
            pltpu.SemaphoreType.DMA,
            pltpu.SemaphoreType.DMA,
        ],
    )
    return f(zt, srcdir, dstdir)


def _sc_gather_body(x_hbm, cdst_hbm, out, cidx_v, rows_v, sem):
    core = lax.axis_index("c")
    sub = lax.axis_index("s")
    wid = sub * 2 + core
    pltpu.sync_copy(cdst_hbm.at[pl.ds(wid * 32, 32)], cidx_v)
    pltpu.async_copy(x_hbm.at[cidx_v], rows_v, sem).wait()
    pltpu.sync_copy(rows_v, out.at[pl.ds(wid * 32, 32)])


def _sc_gather(x, cdst):
    f = pl.kernel(
        _sc_gather_body,
        out_type=jax.ShapeDtypeStruct((CPAD, ZW), jnp.float32),
        mesh=_mesh(),
        compiler_params=_SC_PARAMS,
        scratch_types=[
            pltpu.VMEM((32,), jnp.int32),
            pltpu.VMEM((32, ZW), jnp.float32),
            pltpu.SemaphoreType.DMA,
        ],
    )
    return f(x, cdst)


NB = 8
R = NPAD // NB


def _rows(w):
    return pl.BlockSpec((R, w), lambda i: (i, 0))


def _full(shape):
    return pl.BlockSpec(shape, lambda i: tuple(0 for _ in shape))


def _t0_body(opf, ids, cf2, dg2,
             embp, wof, wem, t5, bd_pre1a, bd_pre2, pb1, pb2_5,
             bd_g1a, bd_g1b,
             x_o, zt_o, cf_o, dis_o):
    deg = 1.0 + dg2[0][:, 0:1] + dg2[1][:, 0:1]
    dis = lax.rsqrt(deg)
    cf = 100.0 * (cf2[0] + cf2[1])
    oh = jnp.where(ids[...] == lax.broadcasted_iota(jnp.int32, (R, 128), 1),
                   1.0, 0.0).astype(jnp.float32)
    e32 = jnp.dot(oh, embp[...], preferred_element_type=jnp.float32)
    p = (jnp.dot(opf[...], wof[...], preferred_element_type=jnp.float32)
         + jnp.dot(e32, wem[...], preferred_element_type=jnp.float32)
         + pb1[...])
    h = _leaky(jnp.dot(cf, bd_pre1a[...], preferred_element_type=jnp.float32)
               + jnp.dot(p, t5[...], preferred_element_type=jnp.float32))
    x = _leaky(jnp.dot(h, bd_pre2[...], preferred_element_type=jnp.float32)
               + pb2_5[...])
    z1 = (jnp.dot(cf, bd_g1a[...], preferred_element_type=jnp.float32)
          + jnp.dot(x, bd_g1b[...], preferred_element_type=jnp.float32))
    x_o[...] = x
    zt_o[...] = dis * z1
    cf_o[...] = cf
    dis_o[...] = dis


def _t0(opf, ids, cf2, dg2, embp, wof, wem, t5, bd_pre1a, bd_pre2, pb1, pb2_5,
        bd_g1a, bd_g1b):
    return pl.pallas_call(
        _t0_body,
        grid=(NB,),
        in_specs=[
            _rows(144), _rows(1),
            pl.BlockSpec((2, R, CF_W), lambda i: (0, i, 0)),
            pl.BlockSpec((2, R, DW), lambda i: (0, i, 0)),
            _full((128, 32)), _full((144, 32)), _full((32, 32)),
            _full((32, ZW)), _full((CF_W, ZW)), _full((ZW, ZW)),
            _full((1, 32)), _full((1, ZW)),
            _full((CF_W, ZW)), _full((ZW, ZW)),
        ],
        out_specs=[_rows(ZW), _rows(ZW), _rows(CF_W), _rows(1)],
        out_shape=[
            jax.ShapeDtypeStruct((NPAD, ZW), jnp.float32),
            jax.ShapeDtypeStruct((NPAD, ZW), jnp.float32),
            jax.ShapeDtypeStruct((NPAD, CF_W), jnp.float32),
            jax.ShapeDtypeStruct((NPAD, 1), jnp.float32),
        ],
    )(opf, ids, cf2, dg2, embp, wof, wem, t5,
      bd_pre1a, bd_pre2, pb1, pb2_5, bd_g1a, bd_g1b)


def _t2_body(zt1, agg, x, cf, dis,
             b1_5, bd_w2, b2_5, bd_g2a, bd_g2b,
             x2_o, zt2_o):
    a = dis[...] * (zt1[...] + agg[...][0] + agg[...][1])
    t = (jnp.dot(_leaky(a + b1_5[...]), bd_w2[...],
                 preferred_element_type=jnp.float32) + b2_5[...])
    x2 = x[...] + _leaky(t)
    z2 = (jnp.dot(cf[...], bd_g2a[...], preferred_element_type=jnp.float32)
          + jnp.dot(x2, bd_g2b[...], preferred_element_type=jnp.float32))
    x2_o[...] = x2
    zt2_o[...] = dis[...] * z2


def _t2(zt1, agg, x, cf, dis, b1_5, bd_w2, b2_5, bd_g2a, bd_g2b):
    return pl.pallas_call(
        _t2_body,
        grid=(NB,),
        in_specs=[
            _rows(ZW),
            pl.BlockSpec((2, R, ZW), lambda i: (0, i, 0)),
            _rows(ZW), _rows(CF_W), _rows(1),
            _full((1, ZW)), _full((ZW, ZW)), _full((1, ZW)),
            _full((CF_W, ZW)), _full((ZW, ZW)),
        ],
        out_specs=[_rows(ZW), _rows(ZW)],
        out_shape=[
            jax.ShapeDtypeStruct((NPAD, ZW), jnp.float32),
            jax.ShapeDtypeStruct((NPAD, ZW), jnp.float32),
        ],
    )(zt1, agg, x, cf, dis, b1_5, bd_w2, b2_5, bd_g2a, bd_g2b)


def _t3_body(zt2, agg, x2, dis, b1_5, bd_w2, b2_5, x3_o):
    a = dis[...] * (zt2[...] + agg[...][0] + agg[...][1])
    t = (jnp.dot(_leaky(a + b1_5[...]), bd_w2[...],
                 preferred_element_type=jnp.float32) + b2_5[...])
    x3_o[...] = x2[...] + _leaky(t)


def _t3(zt2, agg, x2, dis, b1_5, bd_w2, b2_5):
    return pl.pallas_call(
        _t3_body,
        grid=(NB,),
        in_specs=[
            _rows(ZW),
            pl.BlockSpec((2, R, ZW), lambda i: (0, i, 0)),
            _rows(ZW), _rows(1),
            _full((1, ZW)), _full((ZW, ZW)), _full((1, ZW)),
        ],
        out_specs=[_rows(ZW)],
        out_shape=[jax.ShapeDtypeStruct((NPAD, ZW), jnp.float32)],
    )(zt2, agg, x2, dis, b1_5, bd_w2, b2_5)


def _t3sel_body(zt2, agg, x2, dis, b1_5, bd_w2, b2_5, xfull, msk, xsel_o):
    a = dis[...] * (zt2[...] + agg[...][0] + agg[...][1])
    t = (jnp.dot(_leaky(a + b1_5[...]), bd_w2[...],
                 preferred_element_type=jnp.float32) + b2_5[...])
    x3 = x2[...] + _leaky(t)
    m = msk[...]
    xsel_o[...] = m * x3 + (1.0 - m) * xfull[...]


def _t3sel(zt2, agg, x2, dis, b1_5, bd_w2, b2_5, xfull, msk):
    return pl.pallas_call(
        _t3sel_body,
        grid=(NB,),
        in_specs=[
            _rows(ZW),
            pl.BlockSpec((2, R, ZW), lambda i: (0, i, 0)),
            _rows(ZW), _rows(1),
            _full((1, ZW)), _full((ZW, ZW)), _full((1, ZW)),
            _rows(ZW), _rows(1),
        ],
        out_specs=[_rows(ZW)],
        out_shape=[jax.ShapeDtypeStruct((NPAD, ZW), jnp.float32)],
    )(zt2, agg, x2, dis, b1_5, bd_w2, b2_5, xfull, msk)


def _t4_body(xsel, gid, cfgx, gidc, bd_pa, bd_pb, bd_pc, bd_ones, bd_p2, out):
    ohg = jnp.where(gid[...] == lax.broadcasted_iota(jnp.int32, (NPAD, 128), 1),
                    1.0, 0.0).astype(jnp.float32)
    dn = (((0,), (0,)), ((), ()))
    pooled = lax.dot_general(ohg, xsel[...], dn,
                             preferred_element_type=jnp.float32)
    counts = lax.dot_general(ohg, jnp.ones((NPAD, 8), jnp.float32), dn,
                             preferred_element_type=jnp.float32)
    ohc = jnp.where(gidc[...] == lax.broadcasted_iota(jnp.int32, (CPAD, 128), 1),
                    1.0, 0.0).astype(jnp.float32)
    pooledc = lax.dot_general(ohc, cfgx[...], dn,
                              preferred_element_type=jnp.float32)
    ps = pooled[0:8, :]
    cnt = counts[0:8, 0:1]
    pc = pooledc[0:8, :]
    mean = ps / jnp.maximum(cnt, 1.0)
    ss = jnp.dot(ps * ps, bd_ones[...], preferred_element_type=jnp.float32)
    l2s = ps * lax.rsqrt(jnp.maximum(ss, 1e-12))
    sc = jnp.dot(pc * pc, bd_ones[...], preferred_element_type=jnp.float32)
    l2c = pc * lax.rsqrt(jnp.maximum(sc, 1e-12))
    o = (jnp.dot(mean, bd_pa[...], preferred_element_type=jnp.float32)
         + jnp.dot(l2s, bd_pb[...], preferred_element_type=jnp.float32)
         + jnp.dot(l2c, bd_pc[...], preferred_element_type=jnp.float32))
    out[...] = jnp.dot(_leaky(o), bd_p2[...],
                       preferred_element_type=jnp.float32)


def _t4(xsel, gid, cfgx, gidc, bd_pa, bd_pb, bd_pc, bd_ones, bd_p2):
    return pl.pallas_call(
        _t4_body,
        grid=(1,),
        in_specs=[
            pl.BlockSpec((NPAD, ZW), lambda i: (0, 0)),
            pl.BlockSpec((NPAD, 1), lambda i: (0, 0)),
            pl.BlockSpec((CPAD, ZW), lambda i: (0, 0)),
            pl.BlockSpec((CPAD, 1), lambda i: (0, 0)),
            _full((ZW, ZW)), _full((ZW, ZW)), _full((ZW, ZW)),
            _full((ZW, ZW)), _full((ZW, K)),
        ],
        out_specs=[pl.BlockSpec((G, K), lambda i: (0, 0))],
        out_shape=[jax.ShapeDtypeStruct((G, K), jnp.float32)],
    )(xsel, gid, cfgx, gidc, bd_pa, bd_pb, bd_pc, bd_ones, bd_p2)[0]


def _bd_place(w, rs, cs, nrows, ncols):
    m = jnp.zeros((nrows, ncols), jnp.float32)
    for c in range(K):
        m = m.at[c * rs:c * rs + w.shape[0], c * cs:c * cs + w.shape[1]].set(w)
    return m


def _pad_edges(edges):
    src = edges[0].astype(jnp.int32)
    dst = edges[1].astype(jnp.int32)
    e_dir = 2 * src.shape[0]
    e_pad = -(-e_dir // 20480) * 20480
    npad = e_pad - e_dir
    padi = (N_OPS + (jnp.arange(npad, dtype=jnp.int32) % 16))
    src_dir = jnp.concatenate([src, dst, padi])
    dst_dir = jnp.concatenate([dst, src, padi])
    return src_dir, dst_dir


def kernel(op_feats, nconfig_feats, emb, pre_W1, pre_b1, pre_W2, pre_b2,
           gc1_W1, gc1_b1, gc1_W2, gc1_b2, gc2_W1, gc2_b1, gc2_W2, gc2_b2,
           post_W1, post_W2, op_ids, selected, feed_edges, sampled_feed_edges,
           config_dst, sampled_config_dst, graph_id_op, graph_id_config):
    f32 = jnp.float32

    opf = jnp.pad(op_feats, ((0, NPAD - N_OPS), (0, 4)))
    ids = jnp.pad(op_ids.astype(jnp.int32), (0, NPAD - N_OPS))[:, None]
    msk = jnp.pad(selected.astype(f32), (0, NPAD - N_OPS))[:, None]
    gid = jnp.pad(graph_id_op.astype(jnp.int32), (0, NPAD - N_OPS),
                  constant_values=127)[:, None]
    gidc = jnp.pad(graph_id_config.astype(jnp.int32), (0, CPAD - NC),
                   constant_values=127)[:, None]
    cfrows = jnp.pad(nconfig_feats, ((0, CPAD - NC), (0, 0), (0, 6)))
    cfrows = jnp.pad(cfrows.reshape(CPAD, 120), ((0, 0), (0, 8)))
    cpadi = N_OPS + (jnp.arange(CPAD - NC, dtype=jnp.int32) % 16)
    cdst_f = jnp.concatenate([config_dst.astype(jnp.int32), cpadi])
    cdst_s = jnp.concatenate([sampled_config_dst.astype(jnp.int32), cpadi])
    src_f, dst_f = _pad_edges(feed_edges)
    src_s, dst_s = _pad_edges(sampled_feed_edges)

    embp = jnp.pad(emb, ((0, 8), (0, 0)))
    w_cf, w_opf, w_emb = pre_W1[:18], pre_W1[18:158], pre_W1[158:190]
    wof = jnp.pad(w_opf, ((0, 4), (0, 0)))
    t5 = jnp.tile(jnp.eye(32, dtype=f32), (1, K))
    bd_pre1a = _bd_place(w_cf, 24, 32, CF_W, ZW)
    bd_pre2 = _bd_place(pre_W2, 32, 32, ZW, ZW)
    pb1 = pre_b1[None, :]
    pb2_5 = jnp.tile(pre_b2, K)[None, :]
    bd_g1a = _bd_place(gc1_W1[:18], 24, 32, CF_W, ZW)
    bd_g1b = _bd_place(gc1_W1[18:50], 32, 32, ZW, ZW)
    bd_g1w2 = _bd_place(gc1_W2, 32, 32, ZW, ZW)
    g1b1_5 = jnp.tile(gc1_b1, K)[None, :]
    g1b2_5 = jnp.tile(gc1_b2, K)[None, :]
    bd_g2a = _bd_place(gc2_W1[:18], 24, 32, CF_W, ZW)
    bd_g2b = _bd_place(gc2_W1[18:50], 32, 32, ZW, ZW)
    bd_g2w2 = _bd_place(gc2_W2, 32, 32, ZW, ZW)
    g2b1_5 = jnp.tile(gc2_b1, K)[None, :]
    g2b2_5 = jnp.tile(gc2_b2, K)[None, :]
    bd_pa = _bd_place(post_W1[0:32], 32, 32, ZW, ZW)
    bd_pb = _bd_place(post_W1[32:64], 32, 32, ZW, ZW)
    bd_pc = _bd_place(post_W1[64:96], 32, 32, ZW, ZW)
    bd_ones = _bd_place(jnp.ones((32, 32), f32), 32, 32, ZW, ZW)
    bd_p2 = _bd_place(post_W2, 32, 1, ZW, K)

    def path(cdst, src_dir, dst_dir, dg2):
        cf2 = _sc_cf(cdst, cfrows)
        x, zt1, cf, dis = _t0(opf, ids, cf2, dg2, embp, wof, w_emb, t5,
                              bd_pre1a, bd_pre2, pb1, pb2_5, bd_g1a, bd_g1b)
        agg1 = _sc_agg(zt1, src_dir, dst_dir)
        x2, zt2 = _t2(zt1, agg1, x, cf, dis, g1b1_5, bd_g1w2, g1b2_5,
                      bd_g2a, bd_g2b)
        agg2 = _sc_agg(zt2, src_dir, dst_dir)
        return zt2, agg2, x2, dis

    dg_all = _sc_deg2(dst_f, dst_s)
    zt2f, agg2f, x2f, disf = path(cdst_f, src_f, dst_f, dg_all[0])
    x_full = _t3(zt2f, agg2f, x2f, disf, g2b1_5, bd_g2w2, g2b2_5)[0]
    zt2s, agg2s, x2s, diss = path(cdst_s, src_s, dst_s, dg_all[1])
    xsel = _t3sel(zt2s, agg2s, x2s, diss, g2b1_5, bd_g2w2, g2b2_5,
                  x_full, msk)[0]
    cfgx = _sc_gather(xsel, cdst_f)
    out = _t4(xsel, gid, cfgx, gidc, bd_pa, bd_pb, bd_pc, bd_ones, bd_p2)
    return out


# speedup vs baseline: 218.1505x; 1.4174x over previous
import functools

import jax
import jax.numpy as jnp
from jax import lax
from jax.experimental import pallas as pl
from jax.experimental.pallas import tpu as pltpu
from jax.experimental.pallas import tpu_sc as plsc

N_OPS = 10000
NC = 1000
K = 5
G = 8
ALPHA = 0.2
NPAD = 10240
CPAD = 1024
EBLK = 128
NW = 32
ROWS_W = NPAD // NW
ROWS_SUB = NPAD // 16
CF_W = 128
ZW = 160
DW = 16


def _leaky(x):
    return jnp.where(x >= 0, x, ALPHA * x)


def _mesh():
    return plsc.VectorSubcoreMesh(core_axis_name="c", subcore_axis_name="s")


_SC_PARAMS = pltpu.CompilerParams(use_tc_tiling_on_sc=False)


def _fill2d(ref, n, m, val, dtype):
    w = 32 if dtype == jnp.bfloat16 else 16
    chunks = m // w
    v = jnp.full((w,), val, dtype)

    def body(t, carry):
        r = t // chunks
        c = (t % chunks) * w
        ref[r, pl.ds(c, w)] = v
        return carry

    lax.fori_loop(0, n * chunks, body, 0)


def _sc_cf_body(cdst_hbm, cfrows_hbm, cf_out, cf_acc, cidx_v, crow_v, zbuf_v):
    core = lax.axis_index("c")
    sub = lax.axis_index("s")
    wid = sub * 2 + core

    _fill2d(zbuf_v, 128, CF_W, 0.0, jnp.float32)
    r0 = sub * ROWS_SUB
    for j in range(ROWS_SUB // 128):
        pltpu.sync_copy(zbuf_v, cf_acc.at[pl.ds(r0 + j * 128, 128)])
    plsc.subcore_barrier()

    pltpu.sync_copy(cdst_hbm.at[pl.ds(wid * 32, 32)], cidx_v)
    pltpu.sync_copy(cfrows_hbm.at[pl.ds(wid * 32, 32)], crow_v)
    pltpu.sync_copy(crow_v, cf_acc.at[cidx_v], add=True)
    plsc.subcore_barrier()

    pltpu.sync_copy(cf_acc.at[pl.ds(r0, ROWS_SUB)],
                    cf_out.at[core, pl.ds(r0, ROWS_SUB)])


def _sc_cf(cdst, cfrows):
    f = pl.kernel(
        _sc_cf_body,
        out_type=jax.ShapeDtypeStruct((2, NPAD, CF_W), jnp.float32),
        mesh=_mesh(),
        compiler_params=_SC_PARAMS,
        scratch_types=[
            pltpu.VMEM_SHARED((NPAD, CF_W), jnp.float32),
            pltpu.VMEM((32,), jnp.int32),
            pltpu.VMEM((32, CF_W), jnp.float32),
            pltpu.VMEM((128, CF_W), jnp.float32),
        ],
    )
    return f(cdst, cfrows)


EBLK_D = 512


def _sc_deg_body(nblk_f, nblk_s, dstf_hbm, dsts_hbm, deg_out,
                 acc_f, acc_s, idx0, idx1, ones_v, sem0, sem1):
    core = lax.axis_index("c")
    sub = lax.axis_index("s")
    wid = sub * 2 + core

    _fill2d(ones_v, EBLK_D // 4, DW, 0.0, jnp.float32)
    r0 = sub * ROWS_SUB
    for j in range(ROWS_SUB // (EBLK_D // 4)):
        pltpu.sync_copy(ones_v.at[pl.ds(0, EBLK_D // 4)],
                        acc_f.at[pl.ds(r0 + j * (EBLK_D // 4), EBLK_D // 4)])
        pltpu.sync_copy(ones_v.at[pl.ds(0, EBLK_D // 4)],
                        acc_s.at[pl.ds(r0 + j * (EBLK_D // 4), EBLK_D // 4)])
    _fill2d(ones_v, EBLK_D, DW, 1.0, jnp.float32)
    plsc.subcore_barrier()

    def run(dst_hbm, acc, nblk):
        ebase = wid * nblk * EBLK_D
        pltpu.sync_copy(dst_hbm.at[pl.ds(ebase, EBLK_D)], idx0)

        def body(j2, carry):
            pltpu.async_copy(ones_v, acc.at[idx0], sem0, add=True)
            o1 = ebase + (2 * j2 + 1) * EBLK_D
            pltpu.sync_copy(dst_hbm.at[pl.ds(o1, EBLK_D)], idx1)
            pltpu.make_async_copy(ones_v, acc.at[idx0], sem0).wait()
            pltpu.async_copy(ones_v, acc.at[idx1], sem1, add=True)

            @pl.when(2 * j2 + 2 < nblk)
            def _():
                o2 = ebase + (2 * j2 + 2) * EBLK_D
                pltpu.sync_copy(dst_hbm.at[pl.ds(o2, EBLK_D)], idx0)

            pltpu.make_async_copy(ones_v, acc.at[idx1], sem1).wait()
            return carry

        lax.fori_loop(0, nblk // 2, body, 0)

    run(dstf_hbm, acc_f, nblk_f)
    run(dsts_hbm, acc_s, nblk_s)
    plsc.subcore_barrier()

    pltpu.sync_copy(acc_f.at[pl.ds(r0, ROWS_SUB)],
                    deg_out.at[0, core, pl.ds(r0, ROWS_SUB)])
    pltpu.sync_copy(acc_s.at[pl.ds(r0, ROWS_SUB)],
                    deg_out.at[1, core, pl.ds(r0, ROWS_SUB)])


def _sc_deg2(dstdir_f, dstdir_s):
    nblk_f = dstdir_f.shape[0] // (NW * EBLK_D)
    nblk_s = dstdir_s.shape[0] // (NW * EBLK_D)
    body = functools.partial(_sc_deg_body, nblk_f, nblk_s)
    f = pl.kernel(
        body,
        out_type=jax.ShapeDtypeStruct((2, 2, NPAD, DW), jnp.float32),
        mesh=_mesh(),
        compiler_params=_SC_PARAMS,
        scratch_types=[
            pltpu.VMEM_SHARED((NPAD, DW), jnp.float32),
            pltpu.VMEM_SHARED((NPAD, DW), jnp.float32),
            pltpu.VMEM((EBLK_D,), jnp.int32),
            pltpu.VMEM((EBLK_D,), jnp.int32),
            pltpu.VMEM((EBLK_D, DW), jnp.float32),
            pltpu.SemaphoreType.DMA,
            pltpu.SemaphoreType.DMA,
        ],
    )
    return f(dstdir_f, dstdir_s)


EBLK_B = 128


def _sc_agg_body(nblk, zt_hbm, src_hbm, dst_hbm, out,
                 acc, sidx0, didx0, rows0, sidx1, didx1, rows1, sem0, sem1):
    core = lax.axis_index("c")
    sub = lax.axis_index("s")
    wid = sub * 2 + core

    _fill2d(rows0, EBLK_B, ZW, 0, jnp.bfloat16)
    r0 = sub * ROWS_SUB
    for j in range(ROWS_SUB // EBLK_B):
        pltpu.sync_copy(rows0, acc.at[pl.ds(r0 + j * EBLK_B, EBLK_B)])
    plsc.subcore_barrier()

    ebase = wid * nblk * EBLK_B

    pltpu.sync_copy(src_hbm.at[pl.ds(ebase, EBLK_B)], sidx0)
    pltpu.sync_copy(dst_hbm.at[pl.ds(ebase, EBLK_B)], didx0)
    pltpu.async_copy(zt_hbm.at[sidx0], rows0, sem0)

    def body(j2, carry):
        o1 = ebase + (2 * j2 + 1) * EBLK_B
        pltpu.sync_copy(src_hbm.at[pl.ds(o1, EBLK_B)], sidx1)
        pltpu.sync_copy(dst_hbm.at[pl.ds(o1, EBLK_B)], didx1)
        pltpu.async_copy(zt_hbm.at[sidx1], rows1, sem1)
        pltpu.make_async_copy(zt_hbm.at[sidx0], rows0, sem0).wait()
        pltpu.sync_copy(rows0, acc.at[didx0], add=True)

        @pl.when(2 * j2 + 2 < nblk)
        def _():
            o2 = ebase + (2 * j2 + 2) * EBLK_B
            pltpu.sync_copy(src_hbm.at[pl.ds(o2, EBLK_B)], sidx0)
            pltpu.sync_copy(dst_hbm.at[pl.ds(o2, EBLK_B)], didx0)
            pltpu.async_copy(zt_hbm.at[sidx0], rows0, sem0)

        pltpu.make_async_copy(zt_hbm.at[sidx1], rows1, sem1).wait()
        pltpu.sync_copy(rows1, acc.at[didx1], add=True)
        return carry

    lax.fori_loop(0, nblk // 2, body, 0)
    plsc.subcore_barrier()

    pltpu.sync_copy(acc.at[pl.ds(r0, ROWS_SUB)],
                    out.at[core, pl.ds(r0, ROWS_SUB)])


def _sc_agg(zt, srcdir, dstdir):
    e_pad = srcdir.shape[0]
    nblk = e_pad // (NW * EBLK_B)
    body = functools.partial(_sc_agg_body, nblk)
    f = pl.kernel(
        body,
        out_type=jax.ShapeDtypeStruct((2, NPAD, ZW), jnp.bfloat16),
        mesh=_mesh(),
        compiler_params=_SC_PARAMS,
        scratch_types=[
            pltpu.VMEM_SHARED((NPAD, ZW), jnp.bfloat16),
            pltpu.VMEM((EBLK_B,), jnp.int32),
            pltpu.VMEM((EBLK_B,), jnp.int32),
            pltpu.VMEM((EBLK_B, ZW), jnp.bfloat16),
            pltpu.VMEM((EBLK_B,), jnp.int32),
            pltpu.VMEM((EBLK_B,), jnp.int32),
            pltpu.VMEM((EBLK_B, ZW), jnp.bfloat16),
            pltpu.SemaphoreType.DMA,
            pltpu.SemaphoreType.DMA,
        ],
    )
    return f(zt, srcdir, dstdir)


def _sc_gather_body(x_hbm, cdst_hbm, out, cidx_v, rows_v, sem):
    core = lax.axis_index("c")
    sub = lax.axis_index("s")
    wid = sub * 2 + core
    pltpu.sync_copy(cdst_hbm.at[pl.ds(wid * 32, 32)], cidx_v)
    pltpu.async_copy(x_hbm.at[cidx_v], rows_v, sem).wait()
    pltpu.sync_copy(rows_v, out.at[pl.ds(wid * 32, 32)])


def _sc_gather(x, cdst):
    f = pl.kernel(
        _sc_gather_body,
        out_type=jax.ShapeDtypeStruct((CPAD, ZW), jnp.float32),
        mesh=_mesh(),
        compiler_params=_SC_PARAMS,
        scratch_types=[
            pltpu.VMEM((32,), jnp.int32),
            pltpu.VMEM((32, ZW), jnp.float32),
            pltpu.SemaphoreType.DMA,
        ],
    )
    return f(x, cdst)


NB = 8
R = NPAD // NB


def _rows(w):
    return pl.BlockSpec((R, w), lambda i: (i, 0))


def _full(shape):
    return pl.BlockSpec(shape, lambda i: tuple(0 for _ in shape))


def _t0_body(opf, ids, cf2, dg2,
             embp, wof, wem, t5, bd_pre1a, bd_pre2, pb1, pb2_5,
             bd_g1a, bd_g1b,
             x_o, zt_o, cf_o, dis_o):
    deg = 1.0 + dg2[0][:, 0:1] + dg2[1][:, 0:1]
    dis = lax.rsqrt(deg)
    cf = 100.0 * (cf2[0] + cf2[1])
    oh = jnp.where(ids[...] == lax.broadcasted_iota(jnp.int32, (R, 128), 1),
                   1.0, 0.0).astype(jnp.float32)
    e32 = jnp.dot(oh, embp[...], preferred_element_type=jnp.float32)
    p = (jnp.dot(opf[...], wof[...], preferred_element_type=jnp.float32)
         + jnp.dot(e32, wem[...], preferred_element_type=jnp.float32)
         + pb1[...])
    h = _leaky(jnp.dot(cf, bd_pre1a[...], preferred_element_type=jnp.float32)
               + jnp.dot(p, t5[...], preferred_element_type=jnp.float32))
    x = _leaky(jnp.dot(h, bd_pre2[...], preferred_element_type=jnp.float32)
               + pb2_5[...])
    z1 = (jnp.dot(cf, bd_g1a[...], preferred_element_type=jnp.float32)
          + jnp.dot(x, bd_g1b[...], preferred_element_type=jnp.float32))
    x_o[...] = x
    zt_o[...] = (dis * z1).astype(jnp.bfloat16)
    cf_o[...] = cf
    dis_o[...] = dis


def _t0(opf, ids, cf2, dg2, embp, wof, wem, t5, bd_pre1a, bd_pre2, pb1, pb2_5,
        bd_g1a, bd_g1b):
    return pl.pallas_call(
        _t0_body,
        grid=(NB,),
        in_specs=[
            _rows(144), _rows(1),
            pl.BlockSpec((2, R, CF_W), lambda i: (0, i, 0)),
            pl.BlockSpec((2, R, DW), lambda i: (0, i, 0)),
            _full((128, 32)), _full((144, 32)), _full((32, 32)),
            _full((32, ZW)), _full((CF_W, ZW)), _full((ZW, ZW)),
            _full((1, 32)), _full((1, ZW)),
            _full((CF_W, ZW)), _full((ZW, ZW)),
        ],
        out_specs=[_rows(ZW), _rows(ZW), _rows(CF_W), _rows(1)],
        out_shape=[
            jax.ShapeDtypeStruct((NPAD, ZW), jnp.float32),
            jax.ShapeDtypeStruct((NPAD, ZW), jnp.bfloat16),
            jax.ShapeDtypeStruct((NPAD, CF_W), jnp.float32),
            jax.ShapeDtypeStruct((NPAD, 1), jnp.float32),
        ],
    )(opf, ids, cf2, dg2, embp, wof, wem, t5,
      bd_pre1a, bd_pre2, pb1, pb2_5, bd_g1a, bd_g1b)


def _t2_body(zt1, agg, x, cf, dis,
             b1_5, bd_w2, b2_5, bd_g2a, bd_g2b,
             x2_o, zt2_o):
    a = dis[...] * (zt1[...].astype(jnp.float32) + agg[...][0].astype(jnp.float32)
                    + agg[...][1].astype(jnp.float32))
    t = (jnp.dot(_leaky(a + b1_5[...]), bd_w2[...],
                 preferred_element_type=jnp.float32) + b2_5[...])
    x2 = x[...] + _leaky(t)
    z2 = (jnp.dot(cf[...], bd_g2a[...], preferred_element_type=jnp.float32)
          + jnp.dot(x2, bd_g2b[...], preferred_element_type=jnp.float32))
    x2_o[...] = x2
    zt2_o[...] = (dis[...] * z2).astype(jnp.bfloat16)


def _t2(zt1, agg, x, cf, dis, b1_5, bd_w2, b2_5, bd_g2a, bd_g2b):
    return pl.pallas_call(
        _t2_body,
        grid=(NB,),
        in_specs=[
            _rows(ZW),
            pl.BlockSpec((2, R, ZW), lambda i: (0, i, 0)),
            _rows(ZW), _rows(CF_W), _rows(1),
            _full((1, ZW)), _full((ZW, ZW)), _full((1, ZW)),
            _full((CF_W, ZW)), _full((ZW, ZW)),
        ],
        out_specs=[_rows(ZW), _rows(ZW)],
        out_shape=[
            jax.ShapeDtypeStruct((NPAD, ZW), jnp.float32),
            jax.ShapeDtypeStruct((NPAD, ZW), jnp.bfloat16),
        ],
    )(zt1, agg, x, cf, dis, b1_5, bd_w2, b2_5, bd_g2a, bd_g2b)


def _t3_body(zt2, agg, x2, dis, b1_5, bd_w2, b2_5, x3_o):
    a = dis[...] * (zt2[...].astype(jnp.float32) + agg[...][0].astype(jnp.float32)
                    + agg[...][1].astype(jnp.float32))
    t = (jnp.dot(_leaky(a + b1_5[...]), bd_w2[...],
                 preferred_element_type=jnp.float32) + b2_5[...])
    x3_o[...] = x2[...] + _leaky(t)


def _t3(zt2, agg, x2, dis, b1_5, bd_w2, b2_5):
    return pl.pallas_call(
        _t3_body,
        grid=(NB,),
        in_specs=[
            _rows(ZW),
            pl.BlockSpec((2, R, ZW), lambda i: (0, i, 0)),
            _rows(ZW), _rows(1),
            _full((1, ZW)), _full((ZW, ZW)), _full((1, ZW)),
        ],
        out_specs=[_rows(ZW)],
        out_shape=[jax.ShapeDtypeStruct((NPAD, ZW), jnp.float32)],
    )(zt2, agg, x2, dis, b1_5, bd_w2, b2_5)


def _t3sel_body(zt2, agg, x2, dis, b1_5, bd_w2, b2_5, xfull, msk, xsel_o):
    a = dis[...] * (zt2[...].astype(jnp.float32) + agg[...][0].astype(jnp.float32)
                    + agg[...][1].astype(jnp.float32))
    t = (jnp.dot(_leaky(a + b1_5[...]), bd_w2[...],
                 preferred_element_type=jnp.float32) + b2_5[...])
    x3 = x2[...] + _leaky(t)
    m = msk[...]
    xsel_o[...] = m * x3 + (1.0 - m) * xfull[...]


def _t3sel(zt2, agg, x2, dis, b1_5, bd_w2, b2_5, xfull, msk):
    return pl.pallas_call(
        _t3sel_body,
        grid=(NB,),
        in_specs=[
            _rows(ZW),
            pl.BlockSpec((2, R, ZW), lambda i: (0, i, 0)),
            _rows(ZW), _rows(1),
            _full((1, ZW)), _full((ZW, ZW)), _full((1, ZW)),
            _rows(ZW), _rows(1),
        ],
        out_specs=[_rows(ZW)],
        out_shape=[jax.ShapeDtypeStruct((NPAD, ZW), jnp.float32)],
    )(zt2, agg, x2, dis, b1_5, bd_w2, b2_5, xfull, msk)


def _t4_body(xsel, gid, cfgx, gidc, bd_pa, bd_pb, bd_pc, bd_ones, bd_p2, out):
    ohg = jnp.where(gid[...] == lax.broadcasted_iota(jnp.int32, (NPAD, 128), 1),
                    1.0, 0.0).astype(jnp.float32)
    dn = (((0,), (0,)), ((), ()))
    pooled = lax.dot_general(ohg, xsel[...], dn,
                             preferred_element_type=jnp.float32)
    counts = lax.dot_general(ohg, jnp.ones((NPAD, 8), jnp.float32), dn,
                             preferred_element_type=jnp.float32)
    ohc = jnp.where(gidc[...] == lax.broadcasted_iota(jnp.int32, (CPAD, 128), 1),
                    1.0, 0.0).astype(jnp.float32)
    pooledc = lax.dot_general(ohc, cfgx[...], dn,
                              preferred_element_type=jnp.float32)
    ps = pooled[0:8, :]
    cnt = counts[0:8, 0:1]
    pc = pooledc[0:8, :]
    mean = ps / jnp.maximum(cnt, 1.0)
    ss = jnp.dot(ps * ps, bd_ones[...], preferred_element_type=jnp.float32)
    l2s = ps * lax.rsqrt(jnp.maximum(ss, 1e-12))
    sc = jnp.dot(pc * pc, bd_ones[...], preferred_element_type=jnp.float32)
    l2c = pc * lax.rsqrt(jnp.maximum(sc, 1e-12))
    o = (jnp.dot(mean, bd_pa[...], preferred_element_type=jnp.float32)
         + jnp.dot(l2s, bd_pb[...], preferred_element_type=jnp.float32)
         + jnp.dot(l2c, bd_pc[...], preferred_element_type=jnp.float32))
    out[...] = jnp.dot(_leaky(o), bd_p2[...],
                       preferred_element_type=jnp.float32)


def _t4(xsel, gid, cfgx, gidc, bd_pa, bd_pb, bd_pc, bd_ones, bd_p2):
    return pl.pallas_call(
        _t4_body,
        grid=(1,),
        in_specs=[
            pl.BlockSpec((NPAD, ZW), lambda i: (0, 0)),
            pl.BlockSpec((NPAD, 1), lambda i: (0, 0)),
            pl.BlockSpec((CPAD, ZW), lambda i: (0, 0)),
            pl.BlockSpec((CPAD, 1), lambda i: (0, 0)),
            _full((ZW, ZW)), _full((ZW, ZW)), _full((ZW, ZW)),
            _full((ZW, ZW)), _full((ZW, K)),
        ],
        out_specs=[pl.BlockSpec((G, K), lambda i: (0, 0))],
        out_shape=[jax.ShapeDtypeStruct((G, K), jnp.float32)],
    )(xsel, gid, cfgx, gidc, bd_pa, bd_pb, bd_pc, bd_ones, bd_p2)[0]


def _bd_place(w, rs, cs, nrows, ncols):
    m = jnp.zeros((nrows, ncols), jnp.float32)
    for c in range(K):
        m = m.at[c * rs:c * rs + w.shape[0], c * cs:c * cs + w.shape[1]].set(w)
    return m


def _pad_edges(edges):
    src = edges[0].astype(jnp.int32)
    dst = edges[1].astype(jnp.int32)
    e_dir = 2 * src.shape[0]
    e_pad = -(-e_dir // 20480) * 20480
    npad = e_pad - e_dir
    padi = (N_OPS + (jnp.arange(npad, dtype=jnp.int32) % 16))
    src_dir = jnp.concatenate([src, dst, padi])
    dst_dir = jnp.concatenate([dst, src, padi])
    return src_dir, dst_dir


def kernel(op_feats, nconfig_feats, emb, pre_W1, pre_b1, pre_W2, pre_b2,
           gc1_W1, gc1_b1, gc1_W2, gc1_b2, gc2_W1, gc2_b1, gc2_W2, gc2_b2,
           post_W1, post_W2, op_ids, selected, feed_edges, sampled_feed_edges,
           config_dst, sampled_config_dst, graph_id_op, graph_id_config):
    f32 = jnp.float32

    opf = jnp.pad(op_feats, ((0, NPAD - N_OPS), (0, 4)))
    ids = jnp.pad(op_ids.astype(jnp.int32), (0, NPAD - N_OPS))[:, None]
    msk = jnp.pad(selected.astype(f32), (0, NPAD - N_OPS))[:, None]
    gid = jnp.pad(graph_id_op.astype(jnp.int32), (0, NPAD - N_OPS),
                  constant_values=127)[:, None]
    gidc = jnp.pad(graph_id_config.astype(jnp.int32), (0, CPAD - NC),
                   constant_values=127)[:, None]
    cfrows = jnp.pad(nconfig_feats, ((0, CPAD - NC), (0, 0), (0, 6)))
    cfrows = jnp.pad(cfrows.reshape(CPAD, 120), ((0, 0), (0, 8)))
    cpadi = N_OPS + (jnp.arange(CPAD - NC, dtype=jnp.int32) % 16)
    cdst_f = jnp.concatenate([config_dst.astype(jnp.int32), cpadi])
    cdst_s = jnp.concatenate([sampled_config_dst.astype(jnp.int32), cpadi])
    src_f, dst_f = _pad_edges(feed_edges)
    src_s, dst_s = _pad_edges(sampled_feed_edges)

    embp = jnp.pad(emb, ((0, 8), (0, 0)))
    w_cf, w_opf, w_emb = pre_W1[:18], pre_W1[18:158], pre_W1[158:190]
    wof = jnp.pad(w_opf, ((0, 4), (0, 0)))
    t5 = jnp.tile(jnp.eye(32, dtype=f32), (1, K))
    bd_pre1a = _bd_place(w_cf, 24, 32, CF_W, ZW)
    bd_pre2 = _bd_place(pre_W2, 32, 32, ZW, ZW)
    pb1 = pre_b1[None, :]
    pb2_5 = jnp.tile(pre_b2, K)[None, :]
    bd_g1a = _bd_place(gc1_W1[:18], 24, 32, CF_W, ZW)
    bd_g1b = _bd_place(gc1_W1[18:50], 32, 32, ZW, ZW)
    bd_g1w2 = _bd_place(gc1_W2, 32, 32, ZW, ZW)
    g1b1_5 = jnp.tile(gc1_b1, K)[None, :]
    g1b2_5 = jnp.tile(gc1_b2, K)[None, :]
    bd_g2a = _bd_place(gc2_W1[:18], 24, 32, CF_W, ZW)
    bd_g2b = _bd_place(gc2_W1[18:50], 32, 32, ZW, ZW)
    bd_g2w2 = _bd_place(gc2_W2, 32, 32, ZW, ZW)
    g2b1_5 = jnp.tile(gc2_b1, K)[None, :]
    g2b2_5 = jnp.tile(gc2_b2, K)[None, :]
    bd_pa = _bd_place(post_W1[0:32], 32, 32, ZW, ZW)
    bd_pb = _bd_place(post_W1[32:64], 32, 32, ZW, ZW)
    bd_pc = _bd_place(post_W1[64:96], 32, 32, ZW, ZW)
    bd_ones = _bd_place(jnp.ones((32, 32), f32), 32, 32, ZW, ZW)
    bd_p2 = _bd_place(post_W2, 32, 1, ZW, K)

    def path(cdst, src_dir, dst_dir, dg2):
        cf2 = _sc_cf(cdst, cfrows)
        x, zt1, cf, dis = _t0(opf, ids, cf2, dg2, embp, wof, w_emb, t5,
                              bd_pre1a, bd_pre2, pb1, pb2_5, bd_g1a, bd_g1b)
        agg1 = _sc_agg(zt1, src_dir, dst_dir)
        x2, zt2 = _t2(zt1, agg1, x, cf, dis, g1b1_5, bd_g1w2, g1b2_5,
                      bd_g2a, bd_g2b)
        agg2 = _sc_agg(zt2, src_dir, dst_dir)
        return zt2, agg2, x2, dis

    dg_all = _sc_deg2(dst_f, dst_s)
    zt2f, agg2f, x2f, disf = path(cdst_f, src_f, dst_f, dg_all[0])
    x_full = _t3(zt2f, agg2f, x2f, disf, g2b1_5, bd_g2w2, g2b2_5)[0]
    zt2s, agg2s, x2s, diss = path(cdst_s, src_s, dst_s, dg_all[1])
    xsel = _t3sel(zt2s, agg2s, x2s, diss, g2b1_5, bd_g2w2, g2b2_5,
                  x_full, msk)[0]
    cfgx = _sc_gather(xsel, cdst_f)
    out = _t4(xsel, gid, cfgx, gidc, bd_pa, bd_pb, bd_pc, bd_ones, bd_p2)
    return out
